# baseline (speedup 1.0000x reference)
import functools

import jax
import jax.numpy as jnp
from jax import lax
from jax.experimental import pallas as pl
from jax.experimental.pallas import tpu as pltpu
from jax.experimental.pallas import tpu_sc as plsc

N_NODES = 10000
N_EDGES = 320000
D_FEAT = 128
D_EDGE = 16
HIDDEN = 256
OUT = 128

NC = 2
NS = 16
NW = NC * NS
EDGES_PER_W = (2 * N_EDGES) // NW
CHUNK = 80
NCHUNK = EDGES_PER_W // CHUNK
D_PACK = D_FEAT // 2


def _sc_gather(nf16, src, tgt):
    mesh = plsc.VectorSubcoreMesh(core_axis_name="c", subcore_axis_name="s")

    @functools.partial(
        pl.kernel,
        mesh=mesh,
        compiler_params=pltpu.CompilerParams(use_tc_tiling_on_sc=False),
        out_type=[
            jax.ShapeDtypeStruct((N_EDGES, D_PACK), jnp.int32),
            jax.ShapeDtypeStruct((N_EDGES, D_PACK), jnp.int32),
        ],
        scratch_types=[
            pltpu.VMEM((EDGES_PER_W,), jnp.int32),
            pltpu.VMEM((2, CHUNK, D_PACK), jnp.int32),
            pltpu.SemaphoreType.DMA,
            pltpu.SemaphoreType.DMA,
            pltpu.SemaphoreType.DMA,
            pltpu.SemaphoreType.DMA,
        ],
    )
    def gather_kernel(nf_hbm, src_hbm, tgt_hbm, srcg_hbm, tgtg_hbm,
                      idx_all, rows, sem_g0, sem_g1, sem_w0, sem_w1):
        wid = lax.axis_index("s") * NC + lax.axis_index("c")
        side = wid // (NW // 2)
        lane = wid % (NW // 2)
        ebase = lane * EDGES_PER_W

        def stage(idx_hbm, out_hbm):
            pltpu.sync_copy(idx_hbm.at[pl.ds(ebase, EDGES_PER_W)], idx_all)

            def start_gather(c, b, sem):
                pltpu.async_copy(
                    nf_hbm.at[idx_all.at[pl.ds(c * CHUNK, CHUNK)]],
                    rows.at[b], sem)

            def start_write(c, b, sem):
                pltpu.async_copy(
                    rows.at[b], out_hbm.at[pl.ds(ebase + c * CHUNK, CHUNK)],
                    sem)

            def wait_gather(c, b, sem):
                pltpu.make_async_copy(
                    nf_hbm.at[idx_all.at[pl.ds(c * CHUNK, CHUNK)]],
                    rows.at[b], sem).wait()

            def wait_write(c, b, sem):
                pltpu.make_async_copy(
                    rows.at[b], out_hbm.at[pl.ds(ebase + c * CHUNK, CHUNK)],
                    sem).wait()

            sems_g = (sem_g0, sem_g1)
            sems_w = (sem_w0, sem_w1)

            def half(c, b):
                @pl.when(c >= 1)
                def _():
                    wait_write(c - 1, 1 - b, sems_w[1 - b])

                @pl.when(c + 1 < NCHUNK)
                def _():
                    start_gather(c + 1, 1 - b, sems_g[1 - b])

                wait_gather(c, b, sems_g[b])
                start_write(c, b, sems_w[b])

            start_gather(0, 0, sems_g[0])

            def body(ii, carry):
                half(2 * ii, 0)
                half(2 * ii + 1, 1)
                return carry

            lax.fori_loop(0, NCHUNK // 2, body, 0)
            wait_write(NCHUNK - 1, 1, sems_w[1])

        @pl.when(side == 0)
        def _():
            stage(src_hbm, srcg_hbm)

        @pl.when(side == 1)
        def _():
            stage(tgt_hbm, tgtg_hbm)

    return gather_kernel(nf16, src, tgt)


BE = 640


def _tc_mlp(srcg, tgtg, ef, w1a, w1b, w1c, b1, w2, b2):
    grid = N_EDGES // BE

    def body(sg_ref, tg_ref, ef_ref, w1a_ref, w1b_ref, w1c_ref, b1_ref,
             w2_ref, b2_ref, o_ref):
        h = jnp.dot(sg_ref[...], w1a_ref[...], preferred_element_type=jnp.float32)
        h = h + jnp.dot(tg_ref[...], w1b_ref[...], preferred_element_type=jnp.float32)
        h = h + jnp.dot(ef_ref[...], w1c_ref[...], preferred_element_type=jnp.float32)
        h = jnp.maximum(h + b1_ref[...], 0.0)
        o_ref[...] = jnp.dot(h.astype(jnp.bfloat16), w2_ref[...],
                             preferred_element_type=jnp.float32) + b2_ref[...]

    return pl.pallas_call(
        body,
        grid=(grid,),
        in_specs=[
            pl.BlockSpec((BE, D_FEAT), lambda i: (i, 0)),
            pl.BlockSpec((BE, D_FEAT), lambda i: (i, 0)),
            pl.BlockSpec((BE, D_EDGE), lambda i: (i, 0)),
            pl.BlockSpec((D_FEAT, HIDDEN), lambda i: (0, 0)),
            pl.BlockSpec((D_FEAT, HIDDEN), lambda i: (0, 0)),
            pl.BlockSpec((D_EDGE, HIDDEN), lambda i: (0, 0)),
            pl.BlockSpec((1, HIDDEN), lambda i: (0, 0)),
            pl.BlockSpec((HIDDEN, OUT), lambda i: (0, 0)),
            pl.BlockSpec((1, OUT), lambda i: (0, 0)),
        ],
        out_specs=pl.BlockSpec((BE, OUT), lambda i: (i, 0)),
        out_shape=jax.ShapeDtypeStruct((N_EDGES, OUT), jnp.float32),
    )(srcg, tgtg, ef, w1a, w1b, w1c, b1, w2, b2)


def kernel(node_features, edge_index, edge_features, W1, b1, W2, b2):
    src = edge_index[0].astype(jnp.int32)
    tgt = edge_index[1].astype(jnp.int32)
    nf16 = node_features.astype(jnp.bfloat16)
    nf_packed = jax.lax.bitcast_convert_type(
        nf16.reshape(N_NODES, D_PACK, 2), jnp.int32)
    srcg_p, tgtg_p = _sc_gather(nf_packed, src, tgt)
    srcg = jax.lax.bitcast_convert_type(srcg_p, jnp.bfloat16).reshape(
        N_EDGES, D_FEAT)
    tgtg = jax.lax.bitcast_convert_type(tgtg_p, jnp.bfloat16).reshape(
        N_EDGES, D_FEAT)
    w1a = W1[:D_FEAT].astype(jnp.bfloat16)
    w1b = W1[D_FEAT:2 * D_FEAT].astype(jnp.bfloat16)
    w1c = W1[2 * D_FEAT:]
    return _tc_mlp(srcg, tgtg, edge_features, w1a, w1b, w1c,
                   b1.reshape(1, HIDDEN), W2.astype(jnp.bfloat16),
                   b2.reshape(1, OUT))


# speedup vs baseline: 3.2437x; 3.2437x over previous
import functools

import jax
import jax.numpy as jnp
import numpy as np
from jax import lax
from jax.experimental import pallas as pl
from jax.experimental.pallas import tpu as pltpu
from jax.experimental.pallas import tpu_sc as plsc

N_NODES = 10000
N_EDGES = 320000
D_FEAT = 128
D_EDGE = 16
HIDDEN = 256
OUT = 128

BE = 800
B2 = BE // 2
NBLK = N_EDGES // BE

_PERM = np.array(
    [32 * (k // 16) + (k % 16) for k in range(64)]
    + [32 * (k // 16) + 16 + (k % 16) for k in range(64)], dtype=np.int32)

NC = 2
NS = 16
NW = NC * NS
EDGES_PER_W = (2 * N_EDGES) // NW
CHUNK = 40
NPAIR = EDGES_PER_W // (2 * CHUNK)
BLKS_PER_W = EDGES_PER_W // BE
PAIRS_PER_BLK = B2 // CHUNK


def _sc_gather_pack(nf, src, tgt):
    mesh = plsc.VectorSubcoreMesh(core_axis_name="c", subcore_axis_name="s")

    @functools.partial(
        pl.kernel,
        mesh=mesh,
        out_type=[
            jax.ShapeDtypeStruct((N_EDGES // 2, D_FEAT), jnp.int32),
            jax.ShapeDtypeStruct((N_EDGES // 2, D_FEAT), jnp.int32),
        ],
        scratch_types=[
            pltpu.VMEM((EDGES_PER_W,), jnp.int32),
            pltpu.VMEM((2, CHUNK, D_FEAT), jnp.int32),
            pltpu.VMEM((2, CHUNK, D_FEAT), jnp.int32),
            pltpu.VMEM((2, CHUNK, D_FEAT), jnp.int32),
            pltpu.SemaphoreType.DMA,
            pltpu.SemaphoreType.DMA,
            pltpu.SemaphoreType.DMA,
            pltpu.SemaphoreType.DMA,
        ],
    )
    def gather_kernel(nf_hbm, src_hbm, tgt_hbm, srcg_hbm, tgtg_hbm,
                      idx_all, buf_a, buf_b, out_pk,
                      sem_g0, sem_g1, sem_w0, sem_w1):
        wid = lax.axis_index("s") * NC + lax.axis_index("c")
        side = wid // (NW // 2)
        lane = wid % (NW // 2)
        ebase = lane * EDGES_PER_W
        rbase = ebase // 2

        sems_g = (sem_g0, sem_g1)
        sems_w = (sem_w0, sem_w1)

        def stage(idx_hbm, out_hbm):
            pltpu.sync_copy(idx_hbm.at[pl.ds(ebase, EDGES_PER_W)], idx_all)

            def offs(m):
                blk = m // PAIRS_PER_BLK
                q = m % PAIRS_PER_BLK
                ea = blk * BE + q * CHUNK
                return ea, ea + B2, blk * B2 + q * CHUNK

            def start_gathers(m, s):
                ea, eb, _ = offs(m)
                pltpu.async_copy(
                    nf_hbm.at[idx_all.at[pl.ds(ea, CHUNK)]],
                    buf_a.at[s], sems_g[s])
                pltpu.async_copy(
                    nf_hbm.at[idx_all.at[pl.ds(eb, CHUNK)]],
                    buf_b.at[s], sems_g[s])

            def wait_gathers(m, s):
                ea, eb, _ = offs(m)
                pltpu.make_async_copy(
                    nf_hbm.at[idx_all.at[pl.ds(ea, CHUNK)]],
                    buf_a.at[s], sems_g[s]).wait()
                pltpu.make_async_copy(
                    nf_hbm.at[idx_all.at[pl.ds(eb, CHUNK)]],
                    buf_b.at[s], sems_g[s]).wait()

            def start_write(m, s):
                _, _, r = offs(m)
                off = pl.multiple_of(rbase + r, 8)
                pltpu.async_copy(
                    out_pk.at[s], out_hbm.at[pl.ds(off, CHUNK)],
                    sems_w[s])

            def wait_write(m, s):
                _, _, r = offs(m)
                off = pl.multiple_of(rbase + r, 8)
                pltpu.make_async_copy(
                    out_pk.at[s], out_hbm.at[pl.ds(off, CHUNK)],
                    sems_w[s]).wait()

            def do_pack(s):
                def edge_body(j, carry):
                    for half, buf in ((0, buf_a), (1, buf_b)):
                        for g in range(4):
                            a = buf[s, j, pl.ds(32 * g, 16)] + jnp.int32(0x8000)
                            b = buf[s, j, pl.ds(32 * g + 16, 16)] + jnp.int32(0x8000)
                            pk = lax.bitwise_or(
                                lax.shift_right_logical(a, 16),
                                lax.bitwise_and(b, jnp.int32(-65536)))
                            out_pk[s, j, pl.ds(64 * half + 16 * g, 16)] = pk
                    return carry
                lax.fori_loop(0, CHUNK, edge_body, 0)

            def half_iter(m, s):
                @pl.when(m + 1 < NPAIR)
                def _():
                    start_gathers(m + 1, 1 - s)

                wait_gathers(m, s)

                @pl.when(m >= 2)
                def _():
                    wait_write(m - 2, s)

                do_pack(s)
                start_write(m, s)

            start_gathers(0, 0)

            def body(t, carry):
                half_iter(2 * t, 0)
                half_iter(2 * t + 1, 1)
                return carry

            lax.fori_loop(0, NPAIR // 2, body, 0)
            wait_write(NPAIR - 2, 0)
            wait_write(NPAIR - 1, 1)

        @pl.when(side == 0)
        def _():
            stage(src_hbm, srcg_hbm)

        @pl.when(side == 1)
        def _():
            stage(tgt_hbm, tgtg_hbm)

    return gather_kernel(nf, src, tgt)


def _unpack_block(x_i32):
    lo = jax.lax.bitcast_convert_type(
        jax.lax.shift_left(x_i32, 16), jnp.float32)
    hi = jax.lax.bitcast_convert_type(
        jnp.bitwise_and(x_i32, jnp.int32(-65536)), jnp.float32)
    top = jnp.concatenate([lo[:, :64], hi[:, :64]], axis=1)
    bot = jnp.concatenate([lo[:, 64:], hi[:, 64:]], axis=1)
    return jnp.concatenate([top, bot], axis=0).astype(jnp.bfloat16)


def _tc_mlp(srcg_pk, tgtg_pk, ef, w1a, w1b, w1c, b1, w2, b2):
    def body(sg_ref, tg_ref, ef_ref, w1a_ref, w1b_ref, w1c_ref, b1_ref,
             w2_ref, b2_ref, o_ref):
        sg = _unpack_block(sg_ref[...])
        tg = _unpack_block(tg_ref[...])
        h = jnp.dot(sg, w1a_ref[...], preferred_element_type=jnp.float32)
        h = h + jnp.dot(tg, w1b_ref[...], preferred_element_type=jnp.float32)
        h = h + jnp.dot(ef_ref[...], w1c_ref[...], preferred_element_type=jnp.float32)
        h = jnp.maximum(h + b1_ref[...], 0.0)
        o_ref[...] = jnp.dot(h.astype(jnp.bfloat16), w2_ref[...],
                             preferred_element_type=jnp.float32) + b2_ref[...]

    return pl.pallas_call(
        body,
        grid=(NBLK,),
        in_specs=[
            pl.BlockSpec((B2, D_FEAT), lambda i: (i, 0)),
            pl.BlockSpec((B2, D_FEAT), lambda i: (i, 0)),
            pl.BlockSpec((BE, D_EDGE), lambda i: (i, 0)),
            pl.BlockSpec((D_FEAT, HIDDEN), lambda i: (0, 0)),
            pl.BlockSpec((D_FEAT, HIDDEN), lambda i: (0, 0)),
            pl.BlockSpec((D_EDGE, HIDDEN), lambda i: (0, 0)),
            pl.BlockSpec((1, HIDDEN), lambda i: (0, 0)),
            pl.BlockSpec((HIDDEN, OUT), lambda i: (0, 0)),
            pl.BlockSpec((1, OUT), lambda i: (0, 0)),
        ],
        out_specs=pl.BlockSpec((BE, OUT), lambda i: (i, 0)),
        out_shape=jax.ShapeDtypeStruct((N_EDGES, OUT), jnp.float32),
    )(srcg_pk, tgtg_pk, ef, w1a, w1b, w1c, b1, w2, b2)


def kernel(node_features, edge_index, edge_features, W1, b1, W2, b2):
    src = edge_index[0].astype(jnp.int32)
    tgt = edge_index[1].astype(jnp.int32)
    nf_i32 = jax.lax.bitcast_convert_type(node_features, jnp.int32)
    srcg_pk, tgtg_pk = _sc_gather_pack(nf_i32, src, tgt)
    perm = jnp.asarray(_PERM)
    w1a = W1[:D_FEAT][perm].astype(jnp.bfloat16)
    w1b = W1[D_FEAT:2 * D_FEAT][perm].astype(jnp.bfloat16)
    w1c = W1[2 * D_FEAT:]
    return _tc_mlp(srcg_pk, tgtg_pk, edge_features, w1a, w1b, w1c,
                   b1.reshape(1, HIDDEN), W2.astype(jnp.bfloat16),
                   b2.reshape(1, OUT))


# speedup vs baseline: 4.1294x; 1.2730x over previous
import functools

import jax
import jax.numpy as jnp
import numpy as np
from jax import lax
from jax.experimental import pallas as pl
from jax.experimental.pallas import tpu as pltpu
from jax.experimental.pallas import tpu_sc as plsc

N_NODES = 10000
N_EDGES = 320000
D_FEAT = 128
D_EDGE = 16
HIDDEN = 256
OUT = 128

BE = 2000
B2 = BE // 2
NBLK = N_EDGES // BE

_PERM = np.array(
    [32 * (k // 16) + (k % 16) for k in range(64)]
    + [32 * (k // 16) + 16 + (k % 16) for k in range(64)], dtype=np.int32)

NC = 2
NS = 16
NW = NC * NS
EDGES_PER_W = (2 * N_EDGES) // NW
CHUNK = 40
NPAIR = EDGES_PER_W // (2 * CHUNK)
BLKS_PER_W = EDGES_PER_W // BE
PAIRS_PER_BLK = B2 // CHUNK


def _sc_gather_pack(nf, src, tgt):
    mesh = plsc.VectorSubcoreMesh(core_axis_name="c", subcore_axis_name="s")

    @functools.partial(
        pl.kernel,
        mesh=mesh,
        out_type=[
            jax.ShapeDtypeStruct((N_EDGES // 2, D_FEAT), jnp.int32),
            jax.ShapeDtypeStruct((N_EDGES // 2, D_FEAT), jnp.int32),
        ],
        scratch_types=[
            pltpu.VMEM((EDGES_PER_W,), jnp.int32),
            pltpu.VMEM((2, CHUNK, D_FEAT), jnp.int32),
            pltpu.VMEM((2, CHUNK, D_FEAT), jnp.int32),
            pltpu.VMEM((2, CHUNK, D_FEAT), jnp.int32),
            pltpu.SemaphoreType.DMA,
            pltpu.SemaphoreType.DMA,
            pltpu.SemaphoreType.DMA,
            pltpu.SemaphoreType.DMA,
        ],
    )
    def gather_kernel(nf_hbm, src_hbm, tgt_hbm, srcg_hbm, tgtg_hbm,
                      idx_all, buf_a, buf_b, out_pk,
                      sem_g0, sem_g1, sem_w0, sem_w1):
        wid = lax.axis_index("s") * NC + lax.axis_index("c")
        side = wid // (NW // 2)
        lane = wid % (NW // 2)
        ebase = lane * EDGES_PER_W
        rbase = ebase // 2

        sems_g = (sem_g0, sem_g1)
        sems_w = (sem_w0, sem_w1)

        def stage(idx_hbm, out_hbm):
            pltpu.sync_copy(idx_hbm.at[pl.ds(ebase, EDGES_PER_W)], idx_all)

            def offs(m):
                blk = m // PAIRS_PER_BLK
                q = m % PAIRS_PER_BLK
                ea = blk * BE + q * CHUNK
                return ea, ea + B2, blk * B2 + q * CHUNK

            def start_gathers(m, s):
                ea, eb, _ = offs(m)
                pltpu.async_copy(
                    nf_hbm.at[idx_all.at[pl.ds(ea, CHUNK)]],
                    buf_a.at[s], sems_g[s])
                pltpu.async_copy(
                    nf_hbm.at[idx_all.at[pl.ds(eb, CHUNK)]],
                    buf_b.at[s], sems_g[s])

            def wait_gathers(m, s):
                ea, eb, _ = offs(m)
                pltpu.make_async_copy(
                    nf_hbm.at[idx_all.at[pl.ds(ea, CHUNK)]],
                    buf_a.at[s], sems_g[s]).wait()
                pltpu.make_async_copy(
                    nf_hbm.at[idx_all.at[pl.ds(eb, CHUNK)]],
                    buf_b.at[s], sems_g[s]).wait()

            def start_write(m, s):
                _, _, r = offs(m)
                off = pl.multiple_of(rbase + r, 8)
                pltpu.async_copy(
                    out_pk.at[s], out_hbm.at[pl.ds(off, CHUNK)],
                    sems_w[s])

            def wait_write(m, s):
                _, _, r = offs(m)
                off = pl.multiple_of(rbase + r, 8)
                pltpu.make_async_copy(
                    out_pk.at[s], out_hbm.at[pl.ds(off, CHUNK)],
                    sems_w[s]).wait()

            def do_pack(s):
                def edge_body(j, carry):
                    for half, buf in ((0, buf_a), (1, buf_b)):
                        for g in range(4):
                            a = buf[s, j, pl.ds(32 * g, 16)] + jnp.int32(0x8000)
                            b = buf[s, j, pl.ds(32 * g + 16, 16)] + jnp.int32(0x8000)
                            pk = lax.bitwise_or(
                                lax.shift_right_logical(a, 16),
                                lax.bitwise_and(b, jnp.int32(-65536)))
                            out_pk[s, j, pl.ds(64 * half + 16 * g, 16)] = pk
                    return carry
                lax.fori_loop(0, CHUNK, edge_body, 0)

            def half_iter(m, s):
                @pl.when(m + 1 < NPAIR)
                def _():
                    start_gathers(m + 1, 1 - s)

                wait_gathers(m, s)

                @pl.when(m >= 2)
                def _():
                    wait_write(m - 2, s)

                do_pack(s)
                start_write(m, s)

            start_gathers(0, 0)

            def body(t, carry):
                half_iter(2 * t, 0)
                half_iter(2 * t + 1, 1)
                return carry

            lax.fori_loop(0, NPAIR // 2, body, 0)
            wait_write(NPAIR - 2, 0)
            wait_write(NPAIR - 1, 1)

        @pl.when(side == 0)
        def _():
            stage(src_hbm, srcg_hbm)

        @pl.when(side == 1)
        def _():
            stage(tgt_hbm, tgtg_hbm)

    return gather_kernel(nf, src, tgt)


def _unpack_block(x_i32):
    lo = jax.lax.bitcast_convert_type(
        jax.lax.shift_left(x_i32, 16), jnp.float32)
    hi = jax.lax.bitcast_convert_type(
        jnp.bitwise_and(x_i32, jnp.int32(-65536)), jnp.float32)
    top = jnp.concatenate([lo[:, :64], hi[:, :64]], axis=1)
    bot = jnp.concatenate([lo[:, 64:], hi[:, 64:]], axis=1)
    return jnp.concatenate([top, bot], axis=0).astype(jnp.bfloat16)


def _tc_mlp(srcg_pk, tgtg_pk, ef, w1a, w1b, w1c, b1, w2, b2):
    def body(sg_ref, tg_ref, ef_ref, w1a_ref, w1b_ref, w1c_ref, b1_ref,
             w2_ref, b2_ref, o_ref):
        sg = _unpack_block(sg_ref[...])
        tg = _unpack_block(tg_ref[...])
        h = jnp.dot(sg, w1a_ref[...], preferred_element_type=jnp.float32)
        h = h + jnp.dot(tg, w1b_ref[...], preferred_element_type=jnp.float32)
        h = h + jnp.dot(ef_ref[...], w1c_ref[...], preferred_element_type=jnp.float32)
        h = jnp.maximum(h + b1_ref[...], 0.0)
        o_ref[...] = jnp.dot(h.astype(jnp.bfloat16), w2_ref[...],
                             preferred_element_type=jnp.float32) + b2_ref[...]

    return pl.pallas_call(
        body,
        grid=(NBLK,),
        in_specs=[
            pl.BlockSpec((B2, D_FEAT), lambda i: (i, 0)),
            pl.BlockSpec((B2, D_FEAT), lambda i: (i, 0)),
            pl.BlockSpec((BE, D_EDGE), lambda i: (i, 0)),
            pl.BlockSpec((D_FEAT, HIDDEN), lambda i: (0, 0)),
            pl.BlockSpec((D_FEAT, HIDDEN), lambda i: (0, 0)),
            pl.BlockSpec((D_EDGE, HIDDEN), lambda i: (0, 0)),
            pl.BlockSpec((1, HIDDEN), lambda i: (0, 0)),
            pl.BlockSpec((HIDDEN, OUT), lambda i: (0, 0)),
            pl.BlockSpec((1, OUT), lambda i: (0, 0)),
        ],
        out_specs=pl.BlockSpec((BE, OUT), lambda i: (i, 0)),
        out_shape=jax.ShapeDtypeStruct((N_EDGES, OUT), jnp.float32),
    )(srcg_pk, tgtg_pk, ef, w1a, w1b, w1c, b1, w2, b2)


def kernel(node_features, edge_index, edge_features, W1, b1, W2, b2):
    src = edge_index[0].astype(jnp.int32)
    tgt = edge_index[1].astype(jnp.int32)
    nf_i32 = jax.lax.bitcast_convert_type(node_features, jnp.int32)
    srcg_pk, tgtg_pk = _sc_gather_pack(nf_i32, src, tgt)
    perm = jnp.asarray(_PERM)
    w1a = W1[:D_FEAT][perm].astype(jnp.bfloat16)
    w1b = W1[D_FEAT:2 * D_FEAT][perm].astype(jnp.bfloat16)
    w1c = W1[2 * D_FEAT:]
    return _tc_mlp(srcg_pk, tgtg_pk, edge_features, w1a, w1b, w1c,
                   b1.reshape(1, HIDDEN), W2.astype(jnp.bfloat16),
                   b2.reshape(1, OUT))


# speedup vs baseline: 4.5708x; 1.1069x over previous
import functools

import jax
import jax.numpy as jnp
import numpy as np
from jax import lax
from jax.experimental import pallas as pl
from jax.experimental.pallas import tpu as pltpu
from jax.experimental.pallas import tpu_sc as plsc

N_NODES = 10000
N_EDGES = 320000
D_FEAT = 128
D_EDGE = 16
HIDDEN = 256
OUT = 128

BE = 4000
B2 = BE // 2
NBLK = N_EDGES // BE

_PERM = np.array(
    [32 * (k // 16) + (k % 16) for k in range(64)]
    + [32 * (k // 16) + 16 + (k % 16) for k in range(64)], dtype=np.int32)

NC = 2
NS = 16
NW = NC * NS
EDGES_PER_W = (2 * N_EDGES) // NW
CHUNK = 40
NPAIR = EDGES_PER_W // (2 * CHUNK)
BLKS_PER_W = EDGES_PER_W // BE
PAIRS_PER_BLK = B2 // CHUNK


def _sc_gather_pack(nf, src, tgt):
    mesh = plsc.VectorSubcoreMesh(core_axis_name="c", subcore_axis_name="s")

    @functools.partial(
        pl.kernel,
        mesh=mesh,
        out_type=[
            jax.ShapeDtypeStruct((N_EDGES // 2, D_FEAT), jnp.int32),
            jax.ShapeDtypeStruct((N_EDGES // 2, D_FEAT), jnp.int32),
        ],
        scratch_types=[
            pltpu.VMEM((EDGES_PER_W,), jnp.int32),
            pltpu.VMEM((2, CHUNK, D_FEAT), jnp.int32),
            pltpu.VMEM((2, CHUNK, D_FEAT), jnp.int32),
            pltpu.VMEM((2, CHUNK, D_FEAT), jnp.int32),
            pltpu.SemaphoreType.DMA,
            pltpu.SemaphoreType.DMA,
            pltpu.SemaphoreType.DMA,
            pltpu.SemaphoreType.DMA,
        ],
    )
    def gather_kernel(nf_hbm, src_hbm, tgt_hbm, srcg_hbm, tgtg_hbm,
                      idx_all, buf_a, buf_b, out_pk,
                      sem_g0, sem_g1, sem_w0, sem_w1):
        wid = lax.axis_index("s") * NC + lax.axis_index("c")
        side = wid // (NW // 2)
        lane = wid % (NW // 2)
        ebase = lane * EDGES_PER_W
        rbase = ebase // 2

        sems_g = (sem_g0, sem_g1)
        sems_w = (sem_w0, sem_w1)

        def stage(idx_hbm, out_hbm):
            pltpu.sync_copy(idx_hbm.at[pl.ds(ebase, EDGES_PER_W)], idx_all)

            def offs(m):
                blk = m // PAIRS_PER_BLK
                q = m % PAIRS_PER_BLK
                ea = blk * BE + q * CHUNK
                return ea, ea + B2, blk * B2 + q * CHUNK

            def start_gathers(m, s):
                ea, eb, _ = offs(m)
                pltpu.async_copy(
                    nf_hbm.at[idx_all.at[pl.ds(ea, CHUNK)]],
                    buf_a.at[s], sems_g[s])
                pltpu.async_copy(
                    nf_hbm.at[idx_all.at[pl.ds(eb, CHUNK)]],
                    buf_b.at[s], sems_g[s])

            def wait_gathers(m, s):
                ea, eb, _ = offs(m)
                pltpu.make_async_copy(
                    nf_hbm.at[idx_all.at[pl.ds(ea, CHUNK)]],
                    buf_a.at[s], sems_g[s]).wait()
                pltpu.make_async_copy(
                    nf_hbm.at[idx_all.at[pl.ds(eb, CHUNK)]],
                    buf_b.at[s], sems_g[s]).wait()

            def start_write(m, s):
                _, _, r = offs(m)
                off = pl.multiple_of(rbase + r, 8)
                pltpu.async_copy(
                    out_pk.at[s], out_hbm.at[pl.ds(off, CHUNK)],
                    sems_w[s])

            def wait_write(m, s):
                _, _, r = offs(m)
                off = pl.multiple_of(rbase + r, 8)
                pltpu.make_async_copy(
                    out_pk.at[s], out_hbm.at[pl.ds(off, CHUNK)],
                    sems_w[s]).wait()

            def do_pack(s):
                def edge_body(j, carry):
                    for half, buf in ((0, buf_a), (1, buf_b)):
                        for g in range(4):
                            a = buf[s, j, pl.ds(32 * g, 16)] + jnp.int32(0x8000)
                            b = buf[s, j, pl.ds(32 * g + 16, 16)] + jnp.int32(0x8000)
                            pk = lax.bitwise_or(
                                lax.shift_right_logical(a, 16),
                                lax.bitwise_and(b, jnp.int32(-65536)))
                            out_pk[s, j, pl.ds(64 * half + 16 * g, 16)] = pk
                    return carry
                lax.fori_loop(0, CHUNK, edge_body, 0)

            def half_iter(m, s):
                @pl.when(m + 1 < NPAIR)
                def _():
                    start_gathers(m + 1, 1 - s)

                wait_gathers(m, s)

                @pl.when(m >= 2)
                def _():
                    wait_write(m - 2, s)

                do_pack(s)
                start_write(m, s)

            start_gathers(0, 0)

            def body(t, carry):
                half_iter(2 * t, 0)
                half_iter(2 * t + 1, 1)
                return carry

            lax.fori_loop(0, NPAIR // 2, body, 0)
            wait_write(NPAIR - 2, 0)
            wait_write(NPAIR - 1, 1)

        @pl.when(side == 0)
        def _():
            stage(src_hbm, srcg_hbm)

        @pl.when(side == 1)
        def _():
            stage(tgt_hbm, tgtg_hbm)

    return gather_kernel(nf, src, tgt)


def _unpack_block(x_i32):
    lo = jax.lax.bitcast_convert_type(
        jax.lax.shift_left(x_i32, 16), jnp.float32)
    hi = jax.lax.bitcast_convert_type(
        jnp.bitwise_and(x_i32, jnp.int32(-65536)), jnp.float32)
    top = jnp.concatenate([lo[:, :64], hi[:, :64]], axis=1)
    bot = jnp.concatenate([lo[:, 64:], hi[:, 64:]], axis=1)
    return jnp.concatenate([top, bot], axis=0).astype(jnp.bfloat16)


def _tc_mlp(srcg_pk, tgtg_pk, ef, w1a, w1b, w1c, b1, w2, b2):
    def body(sg_ref, tg_ref, ef_ref, w1a_ref, w1b_ref, w1c_ref, b1_ref,
             w2_ref, b2_ref, o_ref):
        sg = _unpack_block(sg_ref[...])
        tg = _unpack_block(tg_ref[...])
        h = jnp.dot(sg, w1a_ref[...], preferred_element_type=jnp.float32)
        h = h + jnp.dot(tg, w1b_ref[...], preferred_element_type=jnp.float32)
        h = h + jnp.dot(ef_ref[...], w1c_ref[...], preferred_element_type=jnp.float32)
        h = jnp.maximum(h + b1_ref[...], 0.0)
        o_ref[...] = jnp.dot(h.astype(jnp.bfloat16), w2_ref[...],
                             preferred_element_type=jnp.float32) + b2_ref[...]

    return pl.pallas_call(
        body,
        grid=(NBLK,),
        in_specs=[
            pl.BlockSpec((B2, D_FEAT), lambda i: (i, 0)),
            pl.BlockSpec((B2, D_FEAT), lambda i: (i, 0)),
            pl.BlockSpec((BE, D_EDGE), lambda i: (i, 0)),
            pl.BlockSpec((D_FEAT, HIDDEN), lambda i: (0, 0)),
            pl.BlockSpec((D_FEAT, HIDDEN), lambda i: (0, 0)),
            pl.BlockSpec((D_EDGE, HIDDEN), lambda i: (0, 0)),
            pl.BlockSpec((1, HIDDEN), lambda i: (0, 0)),
            pl.BlockSpec((HIDDEN, OUT), lambda i: (0, 0)),
            pl.BlockSpec((1, OUT), lambda i: (0, 0)),
        ],
        out_specs=pl.BlockSpec((BE, OUT), lambda i: (i, 0)),
        out_shape=jax.ShapeDtypeStruct((N_EDGES, OUT), jnp.float32),
    )(srcg_pk, tgtg_pk, ef, w1a, w1b, w1c, b1, w2, b2)


def kernel(node_features, edge_index, edge_features, W1, b1, W2, b2):
    src = edge_index[0].astype(jnp.int32)
    tgt = edge_index[1].astype(jnp.int32)
    nf_i32 = jax.lax.bitcast_convert_type(node_features, jnp.int32)
    srcg_pk, tgtg_pk = _sc_gather_pack(nf_i32, src, tgt)
    perm = jnp.asarray(_PERM)
    w1a = W1[:D_FEAT][perm].astype(jnp.bfloat16)
    w1b = W1[D_FEAT:2 * D_FEAT][perm].astype(jnp.bfloat16)
    w1c = W1[2 * D_FEAT:]
    return _tc_mlp(srcg_pk, tgtg_pk, edge_features, w1a, w1b, w1c,
                   b1.reshape(1, HIDDEN), W2.astype(jnp.bfloat16),
                   b2.reshape(1, OUT))


# speedup vs baseline: 4.8589x; 1.0630x over previous
import functools

import jax
import jax.numpy as jnp
import numpy as np
from jax import lax
from jax.experimental import pallas as pl
from jax.experimental.pallas import tpu as pltpu
from jax.experimental.pallas import tpu_sc as plsc

N_NODES = 10000
N_EDGES = 320000
D_FEAT = 128
D_EDGE = 16
HIDDEN = 256
OUT = 128

BE = 10000
B2 = BE // 2
NBLK = N_EDGES // BE

_PERM = np.array(
    [32 * (k // 16) + (k % 16) for k in range(64)]
    + [32 * (k // 16) + 16 + (k % 16) for k in range(64)], dtype=np.int32)

NC = 2
NS = 16
NW = NC * NS
EDGES_PER_W = (2 * N_EDGES) // NW
CHUNK = 40
NPAIR = EDGES_PER_W // (2 * CHUNK)
BLKS_PER_W = EDGES_PER_W // BE
PAIRS_PER_BLK = B2 // CHUNK


def _sc_gather_pack(nf, src, tgt):
    mesh = plsc.VectorSubcoreMesh(core_axis_name="c", subcore_axis_name="s")

    @functools.partial(
        pl.kernel,
        mesh=mesh,
        out_type=[
            jax.ShapeDtypeStruct((N_EDGES // 2, D_FEAT), jnp.int32),
            jax.ShapeDtypeStruct((N_EDGES // 2, D_FEAT), jnp.int32),
        ],
        scratch_types=[
            pltpu.VMEM((EDGES_PER_W,), jnp.int32),
            pltpu.VMEM((2, CHUNK, D_FEAT), jnp.int32),
            pltpu.VMEM((2, CHUNK, D_FEAT), jnp.int32),
            pltpu.VMEM((2, CHUNK, D_FEAT), jnp.int32),
            pltpu.SemaphoreType.DMA,
            pltpu.SemaphoreType.DMA,
            pltpu.SemaphoreType.DMA,
            pltpu.SemaphoreType.DMA,
        ],
    )
    def gather_kernel(nf_hbm, src_hbm, tgt_hbm, srcg_hbm, tgtg_hbm,
                      idx_all, buf_a, buf_b, out_pk,
                      sem_g0, sem_g1, sem_w0, sem_w1):
        wid = lax.axis_index("s") * NC + lax.axis_index("c")
        side = wid // (NW // 2)
        lane = wid % (NW // 2)
        ebase = lane * EDGES_PER_W
        rbase = ebase // 2

        sems_g = (sem_g0, sem_g1)
        sems_w = (sem_w0, sem_w1)

        def stage(idx_hbm, out_hbm):
            pltpu.sync_copy(idx_hbm.at[pl.ds(ebase, EDGES_PER_W)], idx_all)

            def offs(m):
                blk = m // PAIRS_PER_BLK
                q = m % PAIRS_PER_BLK
                ea = blk * BE + q * CHUNK
                return ea, ea + B2, blk * B2 + q * CHUNK

            def start_gathers(m, s):
                ea, eb, _ = offs(m)
                pltpu.async_copy(
                    nf_hbm.at[idx_all.at[pl.ds(ea, CHUNK)]],
                    buf_a.at[s], sems_g[s])
                pltpu.async_copy(
                    nf_hbm.at[idx_all.at[pl.ds(eb, CHUNK)]],
                    buf_b.at[s], sems_g[s])

            def wait_gathers(m, s):
                ea, eb, _ = offs(m)
                pltpu.make_async_copy(
                    nf_hbm.at[idx_all.at[pl.ds(ea, CHUNK)]],
                    buf_a.at[s], sems_g[s]).wait()
                pltpu.make_async_copy(
                    nf_hbm.at[idx_all.at[pl.ds(eb, CHUNK)]],
                    buf_b.at[s], sems_g[s]).wait()

            def start_write(m, s):
                _, _, r = offs(m)
                off = pl.multiple_of(rbase + r, 8)
                pltpu.async_copy(
                    out_pk.at[s], out_hbm.at[pl.ds(off, CHUNK)],
                    sems_w[s])

            def wait_write(m, s):
                _, _, r = offs(m)
                off = pl.multiple_of(rbase + r, 8)
                pltpu.make_async_copy(
                    out_pk.at[s], out_hbm.at[pl.ds(off, CHUNK)],
                    sems_w[s]).wait()

            def do_pack(s):
                def edge_body(j, carry):
                    for half, buf in ((0, buf_a), (1, buf_b)):
                        for g in range(4):
                            a = buf[s, j, pl.ds(32 * g, 16)] + jnp.int32(0x8000)
                            b = buf[s, j, pl.ds(32 * g + 16, 16)] + jnp.int32(0x8000)
                            pk = lax.bitwise_or(
                                lax.shift_right_logical(a, 16),
                                lax.bitwise_and(b, jnp.int32(-65536)))
                            out_pk[s, j, pl.ds(64 * half + 16 * g, 16)] = pk
                    return carry
                lax.fori_loop(0, CHUNK, edge_body, 0)

            def half_iter(m, s):
                @pl.when(m + 1 < NPAIR)
                def _():
                    start_gathers(m + 1, 1 - s)

                wait_gathers(m, s)

                @pl.when(m >= 2)
                def _():
                    wait_write(m - 2, s)

                do_pack(s)
                start_write(m, s)

            start_gathers(0, 0)

            def body(t, carry):
                half_iter(2 * t, 0)
                half_iter(2 * t + 1, 1)
                return carry

            lax.fori_loop(0, NPAIR // 2, body, 0)
            wait_write(NPAIR - 2, 0)
            wait_write(NPAIR - 1, 1)

        @pl.when(side == 0)
        def _():
            stage(src_hbm, srcg_hbm)

        @pl.when(side == 1)
        def _():
            stage(tgt_hbm, tgtg_hbm)

    return gather_kernel(nf, src, tgt)


def _unpack_block(x_i32):
    lo = jax.lax.bitcast_convert_type(
        jax.lax.shift_left(x_i32, 16), jnp.float32)
    hi = jax.lax.bitcast_convert_type(
        jnp.bitwise_and(x_i32, jnp.int32(-65536)), jnp.float32)
    top = jnp.concatenate([lo[:, :64], hi[:, :64]], axis=1)
    bot = jnp.concatenate([lo[:, 64:], hi[:, 64:]], axis=1)
    return jnp.concatenate([top, bot], axis=0).astype(jnp.bfloat16)


def _tc_mlp(srcg_pk, tgtg_pk, ef, w1a, w1b, w1c, b1, w2, b2):
    def body(sg_ref, tg_ref, ef_ref, w1a_ref, w1b_ref, w1c_ref, b1_ref,
             w2_ref, b2_ref, o_ref):
        sg = _unpack_block(sg_ref[...])
        tg = _unpack_block(tg_ref[...])
        h = jnp.dot(sg, w1a_ref[...], preferred_element_type=jnp.float32)
        h = h + jnp.dot(tg, w1b_ref[...], preferred_element_type=jnp.float32)
        h = h + jnp.dot(ef_ref[...], w1c_ref[...], preferred_element_type=jnp.float32)
        h = jnp.maximum(h + b1_ref[...], 0.0)
        o_ref[...] = jnp.dot(h.astype(jnp.bfloat16), w2_ref[...],
                             preferred_element_type=jnp.float32) + b2_ref[...]

    return pl.pallas_call(
        body,
        grid=(NBLK,),
        in_specs=[
            pl.BlockSpec((B2, D_FEAT), lambda i: (i, 0)),
            pl.BlockSpec((B2, D_FEAT), lambda i: (i, 0)),
            pl.BlockSpec((BE, D_EDGE), lambda i: (i, 0)),
            pl.BlockSpec((D_FEAT, HIDDEN), lambda i: (0, 0)),
            pl.BlockSpec((D_FEAT, HIDDEN), lambda i: (0, 0)),
            pl.BlockSpec((D_EDGE, HIDDEN), lambda i: (0, 0)),
            pl.BlockSpec((1, HIDDEN), lambda i: (0, 0)),
            pl.BlockSpec((HIDDEN, OUT), lambda i: (0, 0)),
            pl.BlockSpec((1, OUT), lambda i: (0, 0)),
        ],
        out_specs=pl.BlockSpec((BE, OUT), lambda i: (i, 0)),
        out_shape=jax.ShapeDtypeStruct((N_EDGES, OUT), jnp.float32),
    )(srcg_pk, tgtg_pk, ef, w1a, w1b, w1c, b1, w2, b2)


def kernel(node_features, edge_index, edge_features, W1, b1, W2, b2):
    src = edge_index[0].astype(jnp.int32)
    tgt = edge_index[1].astype(jnp.int32)
    nf_i32 = jax.lax.bitcast_convert_type(node_features, jnp.int32)
    srcg_pk, tgtg_pk = _sc_gather_pack(nf_i32, src, tgt)
    perm = jnp.asarray(_PERM)
    w1a = W1[:D_FEAT][perm].astype(jnp.bfloat16)
    w1b = W1[D_FEAT:2 * D_FEAT][perm].astype(jnp.bfloat16)
    w1c = W1[2 * D_FEAT:]
    return _tc_mlp(srcg_pk, tgtg_pk, edge_features, w1a, w1b, w1c,
                   b1.reshape(1, HIDDEN), W2.astype(jnp.bfloat16),
                   b2.reshape(1, OUT))


# speedup vs baseline: 5.1303x; 1.0559x over previous
import functools

import jax
import jax.numpy as jnp
import numpy as np
from jax import lax
from jax.experimental import pallas as pl
from jax.experimental.pallas import tpu as pltpu
from jax.experimental.pallas import tpu_sc as plsc

N_NODES = 10000
N_EDGES = 320000
D_FEAT = 128
D_EDGE = 16
HIDDEN = 256
OUT = 128

P = 2
NP_E = N_EDGES // P
BE = 10000
B2 = BE // 2
NBLK_P = NP_E // BE

_PERM = np.array(
    [32 * (k // 16) + (k % 16) for k in range(64)]
    + [32 * (k // 16) + 16 + (k % 16) for k in range(64)], dtype=np.int32)

NC = 2
NS = 16
NW = NC * NS
EDGES_PER_W = (2 * NP_E) // NW
CHUNK = 40
NPAIR = EDGES_PER_W // (2 * CHUNK)
PAIRS_PER_BLK = B2 // CHUNK

assert EDGES_PER_W % BE == 0
assert B2 % CHUNK == 0 and CHUNK % 8 == 0 and B2 % 8 == 0
assert EDGES_PER_W % (2 * CHUNK) == 0


def _sc_gather_pack(nf, src, tgt):
    mesh = plsc.VectorSubcoreMesh(core_axis_name="c", subcore_axis_name="s")

    @functools.partial(
        pl.kernel,
        mesh=mesh,
        out_type=[
            jax.ShapeDtypeStruct((NP_E // 2, D_FEAT), jnp.int32),
            jax.ShapeDtypeStruct((NP_E // 2, D_FEAT), jnp.int32),
        ],
        scratch_types=[
            pltpu.VMEM((EDGES_PER_W,), jnp.int32),
            pltpu.VMEM((2, CHUNK, D_FEAT), jnp.int32),
            pltpu.VMEM((2, CHUNK, D_FEAT), jnp.int32),
            pltpu.VMEM((2, CHUNK, D_FEAT), jnp.int32),
            pltpu.SemaphoreType.DMA,
            pltpu.SemaphoreType.DMA,
            pltpu.SemaphoreType.DMA,
            pltpu.SemaphoreType.DMA,
        ],
    )
    def gather_kernel(nf_hbm, src_hbm, tgt_hbm, srcg_hbm, tgtg_hbm,
                      idx_all, buf_a, buf_b, out_pk,
                      sem_g0, sem_g1, sem_w0, sem_w1):
        wid = lax.axis_index("s") * NC + lax.axis_index("c")
        side = wid // (NW // 2)
        lane = wid % (NW // 2)
        ebase = lane * EDGES_PER_W
        rbase = ebase // 2

        sems_g = (sem_g0, sem_g1)
        sems_w = (sem_w0, sem_w1)

        def stage(idx_hbm, out_hbm):
            pltpu.sync_copy(idx_hbm.at[pl.ds(ebase, EDGES_PER_W)], idx_all)

            def offs(m):
                blk = m // PAIRS_PER_BLK
                q = m % PAIRS_PER_BLK
                ea = blk * BE + q * CHUNK
                return ea, ea + B2, blk * B2 + q * CHUNK

            def start_gathers(m, s):
                ea, eb, _ = offs(m)
                pltpu.async_copy(
                    nf_hbm.at[idx_all.at[pl.ds(ea, CHUNK)]],
                    buf_a.at[s], sems_g[s])
                pltpu.async_copy(
                    nf_hbm.at[idx_all.at[pl.ds(eb, CHUNK)]],
                    buf_b.at[s], sems_g[s])

            def wait_gathers(m, s):
                ea, eb, _ = offs(m)
                pltpu.make_async_copy(
                    nf_hbm.at[idx_all.at[pl.ds(ea, CHUNK)]],
                    buf_a.at[s], sems_g[s]).wait()
                pltpu.make_async_copy(
                    nf_hbm.at[idx_all.at[pl.ds(eb, CHUNK)]],
                    buf_b.at[s], sems_g[s]).wait()

            def start_write(m, s):
                _, _, r = offs(m)
                off = pl.multiple_of(rbase + r, 8)
                pltpu.async_copy(
                    out_pk.at[s], out_hbm.at[pl.ds(off, CHUNK)],
                    sems_w[s])

            def wait_write(m, s):
                _, _, r = offs(m)
                off = pl.multiple_of(rbase + r, 8)
                pltpu.make_async_copy(
                    out_pk.at[s], out_hbm.at[pl.ds(off, CHUNK)],
                    sems_w[s]).wait()

            def do_pack(s):
                def edge_body(j, carry):
                    for half, buf in ((0, buf_a), (1, buf_b)):
                        for g in range(4):
                            a = buf[s, j, pl.ds(32 * g, 16)] + jnp.int32(0x8000)
                            b = buf[s, j, pl.ds(32 * g + 16, 16)] + jnp.int32(0x8000)
                            pk = lax.bitwise_or(
                                lax.shift_right_logical(a, 16),
                                lax.bitwise_and(b, jnp.int32(-65536)))
                            out_pk[s, j, pl.ds(64 * half + 16 * g, 16)] = pk
                    return carry
                lax.fori_loop(0, CHUNK, edge_body, 0)

            def half_iter(m, s):
                @pl.when(m + 1 < NPAIR)
                def _():
                    start_gathers(m + 1, 1 - s)

                wait_gathers(m, s)

                @pl.when(m >= 2)
                def _():
                    wait_write(m - 2, s)

                do_pack(s)
                start_write(m, s)

            start_gathers(0, 0)

            def body(t, carry):
                half_iter(2 * t, 0)
                half_iter(2 * t + 1, 1)
                return carry

            lax.fori_loop(0, NPAIR // 2, body, 0)
            if NPAIR % 2:
                half_iter(NPAIR - 1, (NPAIR - 1) % 2)
            wait_write(NPAIR - 2, (NPAIR - 2) % 2)
            wait_write(NPAIR - 1, (NPAIR - 1) % 2)

        @pl.when(side == 0)
        def _():
            stage(src_hbm, srcg_hbm)

        @pl.when(side == 1)
        def _():
            stage(tgt_hbm, tgtg_hbm)

    return gather_kernel(nf, src, tgt)


def _unpack_block(x_i32):
    lo = jax.lax.bitcast_convert_type(
        jax.lax.shift_left(x_i32, 16), jnp.float32)
    hi = jax.lax.bitcast_convert_type(
        jnp.bitwise_and(x_i32, jnp.int32(-65536)), jnp.float32)
    top = jnp.concatenate([lo[:, :64], hi[:, :64]], axis=1)
    bot = jnp.concatenate([lo[:, 64:], hi[:, 64:]], axis=1)
    return jnp.concatenate([top, bot], axis=0).astype(jnp.bfloat16)


def _tc_mlp_part(part, srcg_pk, tgtg_pk, ef, w1a, w1b, w1c, b1, w2, b2,
                 o_prev=None):
    def body(sg_ref, tg_ref, ef_ref, w1a_ref, w1b_ref, w1c_ref, b1_ref,
             w2_ref, b2_ref, *o_refs):
        o_ref = o_refs[-1]
        sg = _unpack_block(sg_ref[...])
        tg = _unpack_block(tg_ref[...])
        h = jnp.dot(sg, w1a_ref[...], preferred_element_type=jnp.float32)
        h = h + jnp.dot(tg, w1b_ref[...], preferred_element_type=jnp.float32)
        h = h + jnp.dot(ef_ref[...], w1c_ref[...], preferred_element_type=jnp.float32)
        h = jnp.maximum(h + b1_ref[...], 0.0)
        o_ref[...] = jnp.dot(h.astype(jnp.bfloat16), w2_ref[...],
                             preferred_element_type=jnp.float32) + b2_ref[...]

    in_specs = [
        pl.BlockSpec((B2, D_FEAT), lambda i: (i, 0)),
        pl.BlockSpec((B2, D_FEAT), lambda i: (i, 0)),
        pl.BlockSpec((BE, D_EDGE), lambda i: (i, 0)),
        pl.BlockSpec((D_FEAT, HIDDEN), lambda i: (0, 0)),
        pl.BlockSpec((D_FEAT, HIDDEN), lambda i: (0, 0)),
        pl.BlockSpec((D_EDGE, HIDDEN), lambda i: (0, 0)),
        pl.BlockSpec((1, HIDDEN), lambda i: (0, 0)),
        pl.BlockSpec((HIDDEN, OUT), lambda i: (0, 0)),
        pl.BlockSpec((1, OUT), lambda i: (0, 0)),
    ]
    args = [srcg_pk, tgtg_pk, ef, w1a, w1b, w1c, b1, w2, b2]
    aliases = {}
    if o_prev is not None:
        in_specs.append(pl.BlockSpec(memory_space=pl.ANY))
        args.append(o_prev)
        aliases = {9: 0}
    return pl.pallas_call(
        body,
        grid=(NBLK_P,),
        in_specs=in_specs,
        out_specs=pl.BlockSpec((BE, OUT), lambda i, _p=part: (_p * NBLK_P + i, 0)),
        out_shape=jax.ShapeDtypeStruct((N_EDGES, OUT), jnp.float32),
        input_output_aliases=aliases,
    )(*args)


def kernel(node_features, edge_index, edge_features, W1, b1, W2, b2):
    src = edge_index[0].astype(jnp.int32)
    tgt = edge_index[1].astype(jnp.int32)
    nf_i32 = jax.lax.bitcast_convert_type(node_features, jnp.int32)

    gathered = []
    for p in range(P):
        gathered.append(_sc_gather_pack(
            nf_i32,
            lax.dynamic_slice_in_dim(src, p * NP_E, NP_E),
            lax.dynamic_slice_in_dim(tgt, p * NP_E, NP_E)))

    perm = jnp.asarray(_PERM)
    w1a = W1[:D_FEAT][perm].astype(jnp.bfloat16)
    w1b = W1[D_FEAT:2 * D_FEAT][perm].astype(jnp.bfloat16)
    w1c = W1[2 * D_FEAT:]
    b1r = b1.reshape(1, HIDDEN)
    w2c = W2.astype(jnp.bfloat16)
    b2r = b2.reshape(1, OUT)

    out = None
    for p in range(P):
        srcg_pk, tgtg_pk = gathered[p]
        ef_p = lax.dynamic_slice_in_dim(edge_features, p * NP_E, NP_E)
        out = _tc_mlp_part(p, srcg_pk, tgtg_pk, ef_p,
                           w1a, w1b, w1c, b1r, w2c, b2r, out)
    return out


# speedup vs baseline: 5.1340x; 1.0007x over previous
import functools

import jax
import jax.numpy as jnp
import numpy as np
from jax import lax
from jax.experimental import pallas as pl
from jax.experimental.pallas import tpu as pltpu
from jax.experimental.pallas import tpu_sc as plsc

N_NODES = 10000
N_EDGES = 320000
D_FEAT = 128
D_EDGE = 16
HIDDEN = 256
OUT = 128

P = 2
NP_E = N_EDGES // P
BE = 10000
B2 = BE // 2
NBLK_P = NP_E // BE

_PERM = np.array(
    [32 * (k // 16) + (k % 16) for k in range(64)]
    + [32 * (k // 16) + 16 + (k % 16) for k in range(64)], dtype=np.int32)

NC = 2
NS = 16
NW = NC * NS
EDGES_PER_W = (2 * NP_E) // NW
CHUNK = 40
NPAIR = EDGES_PER_W // (2 * CHUNK)
PAIRS_PER_BLK = B2 // CHUNK

assert EDGES_PER_W % BE == 0
assert B2 % CHUNK == 0 and CHUNK % 8 == 0 and B2 % 8 == 0
assert EDGES_PER_W % (2 * CHUNK) == 0


def _sc_gather_pack(nf, src, tgt):
    mesh = plsc.VectorSubcoreMesh(core_axis_name="c", subcore_axis_name="s")

    @functools.partial(
        pl.kernel,
        mesh=mesh,
        out_type=[
            jax.ShapeDtypeStruct((NP_E // 2, D_FEAT), jnp.int32),
            jax.ShapeDtypeStruct((NP_E // 2, D_FEAT), jnp.int32),
        ],
        scratch_types=[
            pltpu.VMEM((EDGES_PER_W,), jnp.int32),
            pltpu.VMEM((2, CHUNK, D_FEAT), jnp.int32),
            pltpu.VMEM((2, CHUNK, D_FEAT), jnp.int32),
            pltpu.VMEM((2, CHUNK, D_FEAT), jnp.int32),
            pltpu.SemaphoreType.DMA,
            pltpu.SemaphoreType.DMA,
            pltpu.SemaphoreType.DMA,
            pltpu.SemaphoreType.DMA,
        ],
    )
    def gather_kernel(nf_hbm, src_hbm, tgt_hbm, srcg_hbm, tgtg_hbm,
                      idx_all, buf_a, buf_b, out_pk,
                      sem_g0, sem_g1, sem_w0, sem_w1):
        wid = lax.axis_index("s") * NC + lax.axis_index("c")
        side = wid // (NW // 2)
        lane = wid % (NW // 2)
        ebase = lane * EDGES_PER_W
        rbase = lane * EDGES_PER_W // 2

        sems_g = (sem_g0, sem_g1)
        sems_w = (sem_w0, sem_w1)

        def stage(idx_hbm, out_hbm):
            pltpu.sync_copy(idx_hbm.at[pl.ds(ebase, EDGES_PER_W)], idx_all)

            def offs(m):
                blk = m // PAIRS_PER_BLK
                q = m % PAIRS_PER_BLK
                ea = blk * BE + q * CHUNK
                return ea, ea + B2, blk * B2 + q * CHUNK

            def start_gathers(m, s):
                ea, eb, _ = offs(m)
                pltpu.async_copy(
                    nf_hbm.at[idx_all.at[pl.ds(ea, CHUNK)]],
                    buf_a.at[s], sems_g[s])
                pltpu.async_copy(
                    nf_hbm.at[idx_all.at[pl.ds(eb, CHUNK)]],
                    buf_b.at[s], sems_g[s])

            def wait_gathers(m, s):
                ea, eb, _ = offs(m)
                pltpu.make_async_copy(
                    nf_hbm.at[idx_all.at[pl.ds(ea, CHUNK)]],
                    buf_a.at[s], sems_g[s]).wait()
                pltpu.make_async_copy(
                    nf_hbm.at[idx_all.at[pl.ds(eb, CHUNK)]],
                    buf_b.at[s], sems_g[s]).wait()

            def start_write(m, s):
                _, _, r = offs(m)
                off = pl.multiple_of(rbase + r, 8)
                pltpu.async_copy(
                    out_pk.at[s], out_hbm.at[pl.ds(off, CHUNK)],
                    sems_w[s])

            def wait_write(m, s):
                _, _, r = offs(m)
                off = pl.multiple_of(rbase + r, 8)
                pltpu.make_async_copy(
                    out_pk.at[s], out_hbm.at[pl.ds(off, CHUNK)],
                    sems_w[s]).wait()

            def do_pack(s):
                def edge_body(j, carry):
                    for half, buf in ((0, buf_a), (1, buf_b)):
                        for g in range(4):
                            a = buf[s, j, pl.ds(32 * g, 16)] + jnp.int32(0x8000)
                            b = buf[s, j, pl.ds(32 * g + 16, 16)] + jnp.int32(0x8000)
                            pk = lax.bitwise_or(
                                lax.shift_right_logical(a, 16),
                                lax.bitwise_and(b, jnp.int32(-65536)))
                            out_pk[s, j, pl.ds(64 * half + 16 * g, 16)] = pk
                    return carry
                lax.fori_loop(0, CHUNK, edge_body, 0)

            def half_iter(m, s):
                @pl.when(m + 1 < NPAIR)
                def _():
                    start_gathers(m + 1, 1 - s)

                wait_gathers(m, s)

                @pl.when(m >= 2)
                def _():
                    wait_write(m - 2, s)

                do_pack(s)
                start_write(m, s)

            start_gathers(0, 0)

            def body(t, carry):
                half_iter(2 * t, 0)
                half_iter(2 * t + 1, 1)
                return carry

            lax.fori_loop(0, NPAIR // 2, body, 0)
            if NPAIR % 2:
                half_iter(NPAIR - 1, (NPAIR - 1) % 2)
            wait_write(NPAIR - 2, (NPAIR - 2) % 2)
            wait_write(NPAIR - 1, (NPAIR - 1) % 2)

        @pl.when(side == 0)
        def _():
            stage(src_hbm, srcg_hbm)

        @pl.when(side == 1)
        def _():
            stage(tgt_hbm, tgtg_hbm)

    return gather_kernel(nf, src, tgt)


def _unpack_block(x_i32):
    lo = jax.lax.bitcast_convert_type(
        jax.lax.shift_left(x_i32, 16), jnp.float32)
    hi = jax.lax.bitcast_convert_type(
        jnp.bitwise_and(x_i32, jnp.int32(-65536)), jnp.float32)
    top = jnp.concatenate([lo[:, :64], hi[:, :64]], axis=1)
    bot = jnp.concatenate([lo[:, 64:], hi[:, 64:]], axis=1)
    return jnp.concatenate([top, bot], axis=0).astype(jnp.bfloat16)


def _tc_mlp_part(part, srcg_pk, tgtg_pk, ef, w1a, w1b, w1c, b1, w2, b2,
                 o_prev=None):
    def body(sg_ref, tg_ref, ef_ref, w1a_ref, w1b_ref, w1c_ref, b1_ref,
             w2_ref, b2_ref, *o_refs):
        o_ref = o_refs[-1]
        sg = _unpack_block(sg_ref[...])
        tg = _unpack_block(tg_ref[...])
        h = jnp.dot(sg, w1a_ref[...], preferred_element_type=jnp.float32)
        h = h + jnp.dot(tg, w1b_ref[...], preferred_element_type=jnp.float32)
        h = h + jnp.dot(ef_ref[...], w1c_ref[...], preferred_element_type=jnp.float32)
        h = jnp.maximum(h + b1_ref[...], 0.0)
        o_ref[...] = jnp.dot(h.astype(jnp.bfloat16), w2_ref[...],
                             preferred_element_type=jnp.float32) + b2_ref[...]

    in_specs = [
        pl.BlockSpec((B2, D_FEAT), lambda i: (i, 0)),
        pl.BlockSpec((B2, D_FEAT), lambda i: (i, 0)),
        pl.BlockSpec((BE, D_EDGE), lambda i: (i, 0)),
        pl.BlockSpec((D_FEAT, HIDDEN), lambda i: (0, 0)),
        pl.BlockSpec((D_FEAT, HIDDEN), lambda i: (0, 0)),
        pl.BlockSpec((D_EDGE, HIDDEN), lambda i: (0, 0)),
        pl.BlockSpec((1, HIDDEN), lambda i: (0, 0)),
        pl.BlockSpec((HIDDEN, OUT), lambda i: (0, 0)),
        pl.BlockSpec((1, OUT), lambda i: (0, 0)),
    ]
    args = [srcg_pk, tgtg_pk, ef, w1a, w1b, w1c, b1, w2, b2]
    aliases = {}
    if o_prev is not None:
        in_specs.append(pl.BlockSpec(memory_space=pl.ANY))
        args.append(o_prev)
        aliases = {9: 0}
    return pl.pallas_call(
        body,
        grid=(NBLK_P,),
        in_specs=in_specs,
        out_specs=pl.BlockSpec((BE, OUT), lambda i, _p=part: (_p * NBLK_P + i, 0)),
        out_shape=jax.ShapeDtypeStruct((N_EDGES, OUT), jnp.float32),
        input_output_aliases=aliases,
    )(*args)


def kernel(node_features, edge_index, edge_features, W1, b1, W2, b2):
    src = edge_index[0].astype(jnp.int32)
    tgt = edge_index[1].astype(jnp.int32)
    nf_i32 = jax.lax.bitcast_convert_type(node_features, jnp.int32)

    gathered = []
    for p in range(P):
        gathered.append(_sc_gather_pack(
            nf_i32,
            lax.dynamic_slice_in_dim(src, p * NP_E, NP_E),
            lax.dynamic_slice_in_dim(tgt, p * NP_E, NP_E)))

    perm = jnp.asarray(_PERM)
    w1a = W1[:D_FEAT][perm].astype(jnp.bfloat16)
    w1b = W1[D_FEAT:2 * D_FEAT][perm].astype(jnp.bfloat16)
    w1c = W1[2 * D_FEAT:]
    b1r = b1.reshape(1, HIDDEN)
    w2c = W2.astype(jnp.bfloat16)
    b2r = b2.reshape(1, OUT)

    out = None
    for p in range(P):
        srcg_pk, tgtg_pk = gathered[p]
        ef_p = lax.dynamic_slice_in_dim(edge_features, p * NP_E, NP_E)
        out = _tc_mlp_part(p, srcg_pk, tgtg_pk, ef_p,
                           w1a, w1b, w1c, b1r, w2c, b2r, out)
    return out
